# Initial kernel scaffold; baseline (speedup 1.0000x reference)
import jax
import jax.numpy as jnp
from jax.experimental import pallas as pl


def kernel(user_ids, item_ids, class_ids, user_table, item_table, image_table, text_table, class_table, bias):
    raise NotImplementedError("write your pallas kernel here")


# speedup vs baseline: 1.5242x; 1.5242x over previous
import functools

import jax
import jax.numpy as jnp
from jax import lax
from jax.experimental import pallas as pl
from jax.experimental.pallas import tpu as pltpu
from jax.experimental.pallas import tpu_sc as plsc

B = 16384
NC = 2
NS = 16
NW = NC * NS
CHUNK = B // NW
NJ = CHUNK // 128
L = 16

_mesh = plsc.VectorSubcoreMesh(core_axis_name="c", subcore_axis_name="s")


@functools.partial(
    pl.kernel,
    mesh=_mesh,
    out_type=jax.ShapeDtypeStruct((NW, NJ, 128), jnp.float32),
    scratch_types=[
        pltpu.VMEM((NJ, 128), jnp.int32),
        pltpu.VMEM((NJ, 128), jnp.int32),
        pltpu.VMEM((NJ, 128), jnp.int32),
        pltpu.VMEM((NJ, 128), jnp.float32),
        pltpu.VMEM((NJ, 128), jnp.float32),
        pltpu.VMEM((NJ, 128), jnp.float32),
        pltpu.VMEM((NJ, 128), jnp.float32),
        pltpu.VMEM((NJ, 128), jnp.float32),
        pltpu.VMEM((NJ, 128), jnp.float32),
        pltpu.VMEM((L,), jnp.float32),
        pltpu.SemaphoreType.DMA,
    ],
)
def _fol_sc(uid_hbm, iid_hbm, cid_hbm, ut_hbm, it_hbm, gt_hbm, tt_hbm,
            ct_hbm, b_hbm, out_hbm,
            uidx, iidx, cidx, ubuf, ibuf, gbuf, tbuf, cbuf, obuf,
            bias_v, sem):
    wid = lax.axis_index("s") * NC + lax.axis_index("c")

    pltpu.sync_copy(uid_hbm.at[wid], uidx)
    pltpu.sync_copy(iid_hbm.at[wid], iidx)
    pltpu.sync_copy(cid_hbm.at[wid], cidx)
    pltpu.sync_copy(b_hbm, bias_v.at[pl.ds(0, 1)])

    handles = []
    for j in range(NJ):
        handles.append(pltpu.async_copy(ut_hbm.at[uidx.at[j]], ubuf.at[j], sem))
        handles.append(pltpu.async_copy(it_hbm.at[iidx.at[j]], ibuf.at[j], sem))
        handles.append(pltpu.async_copy(gt_hbm.at[iidx.at[j]], gbuf.at[j], sem))
        handles.append(pltpu.async_copy(tt_hbm.at[iidx.at[j]], tbuf.at[j], sem))
        handles.append(pltpu.async_copy(ct_hbm.at[cidx.at[j]], cbuf.at[j], sem))
    for h in handles:
        h.wait()

    bias = bias_v[...][0]
    for j in range(NJ):
        for v in range(0, 128, L):
            s = pl.ds(v, L)
            obuf[j, s] = (ubuf[j, s] + ibuf[j, s] + gbuf[j, s]
                          + tbuf[j, s] + cbuf[j, s] + bias)

    pltpu.sync_copy(obuf, out_hbm.at[wid])


def kernel(user_ids, item_ids, class_ids, user_table, item_table,
           image_table, text_table, class_table, bias):
    uid = user_ids.astype(jnp.int32).reshape(NW, NJ, 128)
    iid = item_ids.astype(jnp.int32).reshape(NW, NJ, 128)
    cid = class_ids.astype(jnp.int32).reshape(NW, NJ, 128)
    out = _fol_sc(uid, iid, cid,
                  user_table.reshape(-1), item_table.reshape(-1),
                  image_table.reshape(-1), text_table.reshape(-1),
                  class_table.reshape(-1), bias)
    return out.reshape(B, 1)


# speedup vs baseline: 9.1761x; 6.0203x over previous
import functools

import jax
import jax.numpy as jnp
from jax import lax
from jax.experimental import pallas as pl
from jax.experimental.pallas import tpu as pltpu
from jax.experimental.pallas import tpu_sc as plsc

B = 16384
NC = 2
NS = 16
NW = NC * NS
CHUNK = B // NW
NJ = CHUNK // 128
L = 16

_mesh = plsc.VectorSubcoreMesh(core_axis_name="c", subcore_axis_name="s")


@functools.partial(
    pl.kernel,
    mesh=_mesh,
    out_type=jax.ShapeDtypeStruct((NW, NJ, 128), jnp.float32),
    scratch_types=[
        pltpu.VMEM((NJ, 128), jnp.int32),
        pltpu.VMEM((NJ, 128), jnp.int32),
        pltpu.VMEM((NJ, 128), jnp.int32),
        pltpu.VMEM((NJ, 128), jnp.float32),
        pltpu.VMEM((NJ, 128), jnp.float32),
        pltpu.VMEM((NJ, 128), jnp.float32),
        pltpu.VMEM((NJ, 128), jnp.float32),
        pltpu.VMEM((NJ, 128), jnp.float32),
        pltpu.VMEM((NJ, 128), jnp.float32),
        pltpu.VMEM((L,), jnp.float32),
        pltpu.SemaphoreType.DMA,
    ],
)
def _fol_sc(uid_hbm, iid_hbm, cid_hbm, ut_hbm, it_hbm, gt_hbm, tt_hbm,
            ct_hbm, b_hbm, out_hbm,
            uidx, iidx, cidx, ubuf, ibuf, gbuf, tbuf, cbuf, obuf,
            bias_v, sem):
    wid = lax.axis_index("s") * NC + lax.axis_index("c")

    pltpu.sync_copy(uid_hbm.at[wid], uidx)
    pltpu.sync_copy(iid_hbm.at[wid], iidx)
    pltpu.sync_copy(cid_hbm.at[wid], cidx)
    pltpu.sync_copy(b_hbm, bias_v.at[pl.ds(0, 1)])

    handles = []
    for j in range(NJ):
        handles.append(pltpu.async_copy(ut_hbm.at[0].at[uidx.at[j]], ubuf.at[j], sem))
        handles.append(pltpu.async_copy(it_hbm.at[0].at[iidx.at[j]], ibuf.at[j], sem))
        handles.append(pltpu.async_copy(gt_hbm.at[0].at[iidx.at[j]], gbuf.at[j], sem))
        handles.append(pltpu.async_copy(tt_hbm.at[0].at[iidx.at[j]], tbuf.at[j], sem))
        handles.append(pltpu.async_copy(ct_hbm.at[0].at[cidx.at[j]], cbuf.at[j], sem))
    for h in handles:
        h.wait()

    bias = bias_v[...][0]
    for j in range(NJ):
        for v in range(0, 128, L):
            s = pl.ds(v, L)
            obuf[j, s] = (ubuf[j, s] + ibuf[j, s] + gbuf[j, s]
                          + tbuf[j, s] + cbuf[j, s] + bias)

    pltpu.sync_copy(obuf, out_hbm.at[wid])


def kernel(user_ids, item_ids, class_ids, user_table, item_table,
           image_table, text_table, class_table, bias):
    uid = user_ids.astype(jnp.int32).reshape(NW, NJ, 128)
    iid = item_ids.astype(jnp.int32).reshape(NW, NJ, 128)
    cid = class_ids.astype(jnp.int32).reshape(NW, NJ, 128)
    out = _fol_sc(uid, iid, cid,
                  user_table.reshape(1, -1), item_table.reshape(1, -1),
                  image_table.reshape(1, -1), text_table.reshape(1, -1),
                  class_table.reshape(1, -1), bias)
    return out.reshape(B, 1)


# speedup vs baseline: 9.4300x; 1.0277x over previous
import functools

import jax
import jax.numpy as jnp
from jax import lax
from jax.experimental import pallas as pl
from jax.experimental.pallas import tpu as pltpu
from jax.experimental.pallas import tpu_sc as plsc

B = 16384
NC = 2
NS = 16
NW = NC * NS
CHUNK = B // NW
NJ = CHUNK // 128
L = 16

_mesh = plsc.VectorSubcoreMesh(core_axis_name="c", subcore_axis_name="s")


@functools.partial(
    pl.kernel,
    mesh=_mesh,
    out_type=jax.ShapeDtypeStruct((NW, NJ, 128), jnp.float32),
    scratch_types=[
        pltpu.VMEM((NJ, 128), jnp.int32),
        pltpu.VMEM((NJ, 128), jnp.int32),
        pltpu.VMEM((NJ, 128), jnp.int32),
        pltpu.VMEM((NJ, 128), jnp.float32),
        pltpu.VMEM((NJ, 128), jnp.float32),
        pltpu.VMEM((NJ, 128), jnp.float32),
        pltpu.VMEM((NJ, 128), jnp.float32),
        pltpu.VMEM((NJ, 128), jnp.float32),
        pltpu.VMEM((NJ, 128), jnp.float32),
        pltpu.VMEM((L,), jnp.float32),
        pltpu.SemaphoreType.DMA,
        pltpu.SemaphoreType.DMA,
        pltpu.SemaphoreType.DMA,
        pltpu.SemaphoreType.DMA,
    ],
)
def _fol_sc(uid_hbm, iid_hbm, cid_hbm, ut_hbm, it_hbm, gt_hbm, tt_hbm,
            ct_hbm, b_hbm, out_hbm,
            uidx, iidx, cidx, ubuf, ibuf, gbuf, tbuf, cbuf, obuf,
            bias_v, sem, sem_u, sem_i, sem_c):
    wid = lax.axis_index("s") * NC + lax.axis_index("c")

    hu = pltpu.async_copy(uid_hbm.at[wid], uidx, sem_u)
    hi = pltpu.async_copy(iid_hbm.at[wid], iidx, sem_i)
    hc = pltpu.async_copy(cid_hbm.at[wid], cidx, sem_c)
    hb = pltpu.async_copy(b_hbm, bias_v.at[pl.ds(0, 1)], sem)

    handles = [hb]
    hu.wait()
    for j in range(NJ):
        handles.append(pltpu.async_copy(ut_hbm.at[0].at[uidx.at[j]], ubuf.at[j], sem))
    hi.wait()
    for j in range(NJ):
        handles.append(pltpu.async_copy(it_hbm.at[0].at[iidx.at[j]], ibuf.at[j], sem))
        handles.append(pltpu.async_copy(gt_hbm.at[0].at[iidx.at[j]], gbuf.at[j], sem))
        handles.append(pltpu.async_copy(tt_hbm.at[0].at[iidx.at[j]], tbuf.at[j], sem))
    hc.wait()
    for j in range(NJ):
        handles.append(pltpu.async_copy(ct_hbm.at[0].at[cidx.at[j]], cbuf.at[j], sem))
    for h in handles:
        h.wait()

    bias = bias_v[...][0]
    for j in range(NJ):
        for v in range(0, 128, L):
            s = pl.ds(v, L)
            obuf[j, s] = (ubuf[j, s] + ibuf[j, s] + gbuf[j, s]
                          + tbuf[j, s] + cbuf[j, s] + bias)

    pltpu.sync_copy(obuf, out_hbm.at[wid])


def kernel(user_ids, item_ids, class_ids, user_table, item_table,
           image_table, text_table, class_table, bias):
    uid = user_ids.astype(jnp.int32).reshape(NW, NJ, 128)
    iid = item_ids.astype(jnp.int32).reshape(NW, NJ, 128)
    cid = class_ids.astype(jnp.int32).reshape(NW, NJ, 128)
    out = _fol_sc(uid, iid, cid,
                  user_table.reshape(1, -1), item_table.reshape(1, -1),
                  image_table.reshape(1, -1), text_table.reshape(1, -1),
                  class_table.reshape(1, -1), bias)
    return out.reshape(B, 1)
